# baseline (speedup 1.0000x reference)
import functools

import jax
import jax.numpy as jnp
from jax import lax
from jax.experimental import pallas as pl
from jax.experimental.pallas import tpu as pltpu
from jax.experimental.pallas import tpu_sc as plsc

MODEL_DIM = 2048
NUM_EXPERTS = 16
TOP_K = 2
TILE = 2048
N_SUBCORES = 32
LANES = 16


def _matmul_body(x_ref, w_ref, logits_ref):
    logits_ref[...] = jax.lax.dot_general(
        x_ref[...], w_ref[...], (((1,), (1,)), ((), ())),
        preferred_element_type=jnp.float32)


def _gate_logits(x, W):
    n_tokens = x.shape[0]
    return pl.pallas_call(
        _matmul_body,
        grid=(n_tokens // TILE,),
        in_specs=[
            pl.BlockSpec((TILE, MODEL_DIM), lambda i: (i, 0)),
            pl.BlockSpec((NUM_EXPERTS, MODEL_DIM), lambda i: (0, 0)),
        ],
        out_specs=pl.BlockSpec((TILE, NUM_EXPERTS), lambda i: (i, 0)),
        out_shape=jax.ShapeDtypeStruct((n_tokens, NUM_EXPERTS), jnp.float32),
        compiler_params=pltpu.CompilerParams(
            dimension_semantics=("arbitrary",),
        ),
    )(x, W)


def _topk_body(rows_per_sub, logits_hbm, wts_hbm, idx_hbm,
               lg_v, wts_v, idx_v):
    wid = lax.axis_index("s") * 2 + lax.axis_index("c")
    base = wid * rows_per_sub
    pltpu.sync_copy(
        logits_hbm.at[pl.ds(base * NUM_EXPERTS, rows_per_sub * NUM_EXPERTS)],
        lg_v)

    n_blocks = rows_per_sub // LANES

    def block(b, carry):
        rows = b * LANES + lax.iota(jnp.int32, LANES)
        m1 = plsc.load_gather(lg_v, [rows * NUM_EXPERTS])
        i1 = jnp.zeros((LANES,), jnp.int32)
        m2 = jnp.full((LANES,), -jnp.inf, jnp.float32)
        i2 = jnp.zeros((LANES,), jnp.int32)
        for e in range(1, NUM_EXPERTS):
            v = plsc.load_gather(lg_v, [rows * NUM_EXPERTS + e])
            ev = jnp.full((LANES,), e, jnp.int32)
            gt1 = v > m1
            gt2 = v > m2
            m2 = jnp.where(gt1, m1, jnp.where(gt2, v, m2))
            i2 = jnp.where(gt1, i1, jnp.where(gt2, ev, i2))
            m1 = jnp.where(gt1, v, m1)
            i1 = jnp.where(gt1, ev, i1)
        e2 = jnp.exp(m2 - m1)
        w1 = 1.0 / (1.0 + e2)
        w2 = e2 * w1
        pos = rows * TOP_K
        plsc.store_scatter(wts_v, [pos], w1)
        plsc.store_scatter(wts_v, [pos + 1], w2)
        plsc.store_scatter(idx_v, [pos], i1)
        plsc.store_scatter(idx_v, [pos + 1], i2)
        return carry

    lax.fori_loop(0, n_blocks, block, 0)
    pltpu.sync_copy(wts_v,
                    wts_hbm.at[pl.ds(base * TOP_K, rows_per_sub * TOP_K)])
    pltpu.sync_copy(idx_v,
                    idx_hbm.at[pl.ds(base * TOP_K, rows_per_sub * TOP_K)])


def _topk_softmax(logits):
    n_tokens = logits.shape[0]
    rows_per_sub = n_tokens // N_SUBCORES
    mesh = plsc.VectorSubcoreMesh(core_axis_name="c", subcore_axis_name="s")
    wts_flat, idx_flat = pl.kernel(
        functools.partial(_topk_body, rows_per_sub),
        out_type=[
            jax.ShapeDtypeStruct((n_tokens * TOP_K,), jnp.float32),
            jax.ShapeDtypeStruct((n_tokens * TOP_K,), jnp.int32),
        ],
        mesh=mesh,
        scratch_types=[
            pltpu.VMEM((rows_per_sub * NUM_EXPERTS,), jnp.float32),
            pltpu.VMEM((rows_per_sub * TOP_K,), jnp.float32),
            pltpu.VMEM((rows_per_sub * TOP_K,), jnp.int32),
        ],
        compiler_params=pltpu.CompilerParams(needs_layout_passes=False),
    )(logits.reshape(n_tokens * NUM_EXPERTS))
    return (wts_flat.reshape(n_tokens, TOP_K),
            idx_flat.reshape(n_tokens, TOP_K))


@jax.jit
def kernel(x, W):
    logits = _gate_logits(x, W)
    wts, idx = _topk_softmax(logits)
    return wts, idx, logits


# speedup vs baseline: 2.0342x; 2.0342x over previous
import jax
import jax.numpy as jnp
from jax.experimental import pallas as pl
from jax.experimental.pallas import tpu as pltpu

MODEL_DIM = 2048
NUM_EXPERTS = 16
TOP_K = 2
TILE = 2048


def _gate_body(x_ref, w_ref, logits_ref, wts_ref, idx_ref):
    x = x_ref[...]
    w = w_ref[...]
    logits = jax.lax.dot_general(
        x, w, (((1,), (1,)), ((), ())),
        preferred_element_type=jnp.float32)
    logits_ref[...] = logits.T

    lane = jax.lax.broadcasted_iota(jnp.int32, logits.shape, 1)
    m1 = jnp.max(logits, axis=1, keepdims=True)
    i1 = jnp.min(jnp.where(logits == m1, lane, NUM_EXPERTS), axis=1,
                 keepdims=True)
    masked = jnp.where(lane == i1, -jnp.inf, logits)
    m2 = jnp.max(masked, axis=1, keepdims=True)
    i2 = jnp.min(jnp.where(masked == m2, lane, NUM_EXPERTS), axis=1,
                 keepdims=True)
    e = jnp.exp(m2 - m1)
    w1 = 1.0 / (1.0 + e)
    w2 = e * w1
    wts_ref[...] = jnp.concatenate([w1, w2], axis=1).T
    idx_ref[...] = jnp.concatenate([i1, i2], axis=1).T


@jax.jit
def kernel(x, W):
    n_tokens = x.shape[0]
    logits_t, wts_t, idx_t = pl.pallas_call(
        _gate_body,
        grid=(n_tokens // TILE,),
        in_specs=[
            pl.BlockSpec((TILE, MODEL_DIM), lambda i: (i, 0)),
            pl.BlockSpec((NUM_EXPERTS, MODEL_DIM), lambda i: (0, 0)),
        ],
        out_specs=[
            pl.BlockSpec((NUM_EXPERTS, TILE), lambda i: (0, i)),
            pl.BlockSpec((TOP_K, TILE), lambda i: (0, i)),
            pl.BlockSpec((TOP_K, TILE), lambda i: (0, i)),
        ],
        out_shape=[
            jax.ShapeDtypeStruct((NUM_EXPERTS, n_tokens), jnp.float32),
            jax.ShapeDtypeStruct((TOP_K, n_tokens), jnp.float32),
            jax.ShapeDtypeStruct((TOP_K, n_tokens), jnp.int32),
        ],
        compiler_params=pltpu.CompilerParams(
            dimension_semantics=("arbitrary",),
            vmem_limit_bytes=50 * 1024 * 1024,
        ),
    )(x, W)
    return wts_t.T, idx_t.T, logits_t.T


# speedup vs baseline: 2.3297x; 1.1453x over previous
import jax
import jax.numpy as jnp
from jax.experimental import pallas as pl
from jax.experimental.pallas import tpu as pltpu

MODEL_DIM = 2048
NUM_EXPERTS = 16
TOP_K = 2
TILE = 2048


def _gate_body(x_ref, w_ref, logits_ref, wts_ref, idx_ref):
    x = x_ref[...]
    w = w_ref[...]
    logits_t = jax.lax.dot_general(
        w, x, (((1,), (1,)), ((), ())),
        preferred_element_type=jnp.float32)
    logits_ref[...] = logits_t

    expert = jax.lax.broadcasted_iota(jnp.int32, logits_t.shape, 0)
    m1 = jnp.max(logits_t, axis=0, keepdims=True)
    i1 = jnp.min(jnp.where(logits_t == m1, expert, NUM_EXPERTS), axis=0,
                 keepdims=True)
    masked = jnp.where(expert == i1, -jnp.inf, logits_t)
    m2 = jnp.max(masked, axis=0, keepdims=True)
    i2 = jnp.min(jnp.where(masked == m2, expert, NUM_EXPERTS), axis=0,
                 keepdims=True)
    e = jnp.exp(m2 - m1)
    w1 = 1.0 / (1.0 + e)
    w2 = e * w1
    wts_ref[...] = jnp.concatenate([w1, w2], axis=0)
    idx_ref[...] = jnp.concatenate([i1, i2], axis=0)


@jax.jit
def kernel(x, W):
    n_tokens = x.shape[0]
    logits_t, wts_t, idx_t = pl.pallas_call(
        _gate_body,
        grid=(n_tokens // TILE,),
        in_specs=[
            pl.BlockSpec((TILE, MODEL_DIM), lambda i: (i, 0)),
            pl.BlockSpec((NUM_EXPERTS, MODEL_DIM), lambda i: (0, 0)),
        ],
        out_specs=[
            pl.BlockSpec((NUM_EXPERTS, TILE), lambda i: (0, i)),
            pl.BlockSpec((TOP_K, TILE), lambda i: (0, i)),
            pl.BlockSpec((TOP_K, TILE), lambda i: (0, i)),
        ],
        out_shape=[
            jax.ShapeDtypeStruct((NUM_EXPERTS, n_tokens), jnp.float32),
            jax.ShapeDtypeStruct((TOP_K, n_tokens), jnp.float32),
            jax.ShapeDtypeStruct((TOP_K, n_tokens), jnp.int32),
        ],
        compiler_params=pltpu.CompilerParams(
            dimension_semantics=("arbitrary",),
            vmem_limit_bytes=50 * 1024 * 1024,
        ),
    )(x, W)
    return wts_t.T, idx_t.T, logits_t.T


# speedup vs baseline: 2.4132x; 1.0358x over previous
import jax
import jax.numpy as jnp
from jax.experimental import pallas as pl
from jax.experimental.pallas import tpu as pltpu

MODEL_DIM = 2048
NUM_EXPERTS = 16
TOP_K = 2
TILE = 1024


def _gate_body(x_ref, w_ref, logits_ref, wts_ref, idx_ref):
    x = x_ref[...]
    w = w_ref[...]
    logits_t = jax.lax.dot_general(
        w, x, (((1,), (1,)), ((), ())),
        preferred_element_type=jnp.float32)
    logits_ref[...] = logits_t

    expert = jax.lax.broadcasted_iota(jnp.int32, logits_t.shape, 0)
    m1 = jnp.max(logits_t, axis=0, keepdims=True)
    i1 = jnp.min(jnp.where(logits_t == m1, expert, NUM_EXPERTS), axis=0,
                 keepdims=True)
    masked = jnp.where(expert == i1, -jnp.inf, logits_t)
    m2 = jnp.max(masked, axis=0, keepdims=True)
    i2 = jnp.min(jnp.where(masked == m2, expert, NUM_EXPERTS), axis=0,
                 keepdims=True)
    e = jnp.exp(m2 - m1)
    w1 = 1.0 / (1.0 + e)
    w2 = e * w1
    wts_ref[...] = jnp.concatenate([w1, w2], axis=0)
    idx_ref[...] = jnp.concatenate([i1, i2], axis=0)


@jax.jit
def kernel(x, W):
    n_tokens = x.shape[0]
    logits_t, wts_t, idx_t = pl.pallas_call(
        _gate_body,
        grid=(n_tokens // TILE,),
        in_specs=[
            pl.BlockSpec((TILE, MODEL_DIM), lambda i: (i, 0)),
            pl.BlockSpec((NUM_EXPERTS, MODEL_DIM), lambda i: (0, 0)),
        ],
        out_specs=[
            pl.BlockSpec((NUM_EXPERTS, TILE), lambda i: (0, i)),
            pl.BlockSpec((TOP_K, TILE), lambda i: (0, i)),
            pl.BlockSpec((TOP_K, TILE), lambda i: (0, i)),
        ],
        out_shape=[
            jax.ShapeDtypeStruct((NUM_EXPERTS, n_tokens), jnp.float32),
            jax.ShapeDtypeStruct((TOP_K, n_tokens), jnp.float32),
            jax.ShapeDtypeStruct((TOP_K, n_tokens), jnp.int32),
        ],
        compiler_params=pltpu.CompilerParams(
            dimension_semantics=("arbitrary",),
            vmem_limit_bytes=50 * 1024 * 1024,
        ),
    )(x, W)
    return wts_t.T, idx_t.T, logits_t.T
